# baseline (speedup 1.0000x reference)
import functools

import jax
import jax.numpy as jnp
from jax import lax
from jax.experimental import pallas as pl
from jax.experimental.pallas import tpu as pltpu


@functools.lru_cache(maxsize=None)
def _make_tc(batch: int, num_nodes: int, d_emb: int):
    def body(ids_smem, tbl_ref, out_ref, sem):
        def loop(i, _):
            row = ids_smem[i]
            pltpu.make_async_copy(tbl_ref.at[row], out_ref.at[i], sem).start()
            return 0

        lax.fori_loop(0, batch, loop, 0, unroll=8)
        pltpu.make_async_copy(
            tbl_ref.at[pl.ds(0, batch)], out_ref, sem).wait()

    return pl.pallas_call(
        body,
        grid_spec=pltpu.PrefetchScalarGridSpec(
            num_scalar_prefetch=1,
            grid=(1,),
            in_specs=[pl.BlockSpec(memory_space=pltpu.MemorySpace.HBM)],
            out_specs=pl.BlockSpec(memory_space=pltpu.MemorySpace.HBM),
            scratch_shapes=[pltpu.SemaphoreType.DMA],
        ),
        out_shape=jax.ShapeDtypeStruct((batch, d_emb), jnp.float32),
    )


def kernel(ids, E_target, E_context):
    ids = ids.astype(jnp.int32)
    n, d = E_target.shape
    k = _make_tc(ids.shape[0], n, d)
    return k(ids, E_target), k(ids, E_context)


# speedup vs baseline: 1.3201x; 1.3201x over previous
import functools

import jax
import jax.numpy as jnp
from jax import lax
from jax.experimental import pallas as pl
from jax.experimental.pallas import tpu as pltpu
from jax.experimental.pallas import tpu_sc as plsc

LANES = 16
TC_SHARE = 6144


@functools.lru_cache(maxsize=None)
def _make_sc(batch: int, num_nodes: int, d_emb: int):
    info = plsc.get_sparse_core_info()
    nc, ns = info.num_cores, info.num_subcores
    nw = nc * ns
    b_per_w = batch // nw
    mesh = plsc.VectorSubcoreMesh(core_axis_name="c", subcore_axis_name="s")

    @functools.partial(
        pl.kernel,
        mesh=mesh,
        out_type=(
            jax.ShapeDtypeStruct((batch, d_emb), jnp.float32),
            jax.ShapeDtypeStruct((batch, d_emb), jnp.float32),
        ),
        scratch_types=[
            pltpu.VMEM((b_per_w + LANES,), jnp.int32),
            pltpu.VMEM((b_per_w, d_emb), jnp.float32),
            pltpu.SemaphoreType.DMA,
        ],
        compiler_params=pltpu.CompilerParams(use_tc_tiling_on_sc=True),
    )
    def k(ids_hbm, tgt_hbm, ctx_hbm, out_t, out_c, idx_v, rows_v, sem_g):
        wid = lax.axis_index("s") * nc + lax.axis_index("c")
        base = wid * b_per_w
        pltpu.sync_copy(ids_hbm.at[pl.ds(base, b_per_w)],
                        idx_v.at[pl.ds(0, b_per_w)])
        out_slice = pl.ds(base, b_per_w)
        for tbl, out_hbm in ((tgt_hbm, out_t), (ctx_hbm, out_c)):

            def body(i, _):
                row = idx_v[pl.ds(i, LANES)][0]
                pltpu.async_copy(tbl.at[row], rows_v.at[i], sem_g)
                return 0

            lax.fori_loop(0, b_per_w, body, 0)
            pltpu.make_async_copy(
                out_hbm.at[out_slice], rows_v, sem_g).wait()
            pltpu.sync_copy(rows_v, out_hbm.at[out_slice])

    return k


@functools.lru_cache(maxsize=None)
def _make_tc(batch: int, num_nodes: int, d_emb: int):
    def body(ids_smem, tgt_ref, ctx_ref, out_t, out_c, sem):
        for tbl, out in ((tgt_ref, out_t), (ctx_ref, out_c)):

            def loop(i, _):
                row = ids_smem[i]
                pltpu.make_async_copy(tbl.at[row], out.at[i], sem).start()
                return 0

            lax.fori_loop(0, batch, loop, 0, unroll=8)
        pltpu.make_async_copy(
            tgt_ref.at[pl.ds(0, batch)], out_t, sem).wait()
        pltpu.make_async_copy(
            ctx_ref.at[pl.ds(0, batch)], out_c, sem).wait()

    return pl.pallas_call(
        body,
        grid_spec=pltpu.PrefetchScalarGridSpec(
            num_scalar_prefetch=1,
            grid=(1,),
            in_specs=[pl.BlockSpec(memory_space=pltpu.MemorySpace.HBM)] * 2,
            out_specs=[pl.BlockSpec(memory_space=pltpu.MemorySpace.HBM)] * 2,
            scratch_shapes=[pltpu.SemaphoreType.DMA],
        ),
        out_shape=(
            jax.ShapeDtypeStruct((batch, d_emb), jnp.float32),
            jax.ShapeDtypeStruct((batch, d_emb), jnp.float32),
        ),
    )


def kernel(ids, E_target, E_context):
    ids = ids.astype(jnp.int32)
    n, d = E_target.shape
    b = ids.shape[0]
    sc_k = _make_sc(b - TC_SHARE, n, d)
    tc_k = _make_tc(TC_SHARE, n, d)
    t_hi, c_hi = sc_k(ids[TC_SHARE:], E_target, E_context)
    t_lo, c_lo = tc_k(ids[:TC_SHARE], E_target, E_context)
    return (jnp.concatenate([t_lo, t_hi], axis=0),
            jnp.concatenate([c_lo, c_hi], axis=0))


# speedup vs baseline: 1.3206x; 1.0004x over previous
import functools

import jax
import jax.numpy as jnp
from jax import lax
from jax.experimental import pallas as pl
from jax.experimental.pallas import tpu as pltpu
from jax.experimental.pallas import tpu_sc as plsc

LANES = 16
TC_SHARE = 6144


@functools.lru_cache(maxsize=None)
def _make_sc(batch: int, num_nodes: int, d_emb: int):
    info = plsc.get_sparse_core_info()
    nc, ns = info.num_cores, info.num_subcores
    nw = nc * ns
    b_per_w = batch // nw
    mesh = plsc.VectorSubcoreMesh(core_axis_name="c", subcore_axis_name="s")

    @functools.partial(
        pl.kernel,
        mesh=mesh,
        out_type=(
            jax.ShapeDtypeStruct((batch, d_emb), jnp.float32),
            jax.ShapeDtypeStruct((batch, d_emb), jnp.float32),
        ),
        scratch_types=[
            pltpu.VMEM((b_per_w + LANES,), jnp.int32),
            pltpu.VMEM((b_per_w, d_emb), jnp.float32),
            pltpu.SemaphoreType.DMA,
        ],
        compiler_params=pltpu.CompilerParams(use_tc_tiling_on_sc=True),
        cost_estimate=pl.CostEstimate(
            flops=0, transcendentals=0, bytes_accessed=1_000_000_000),
    )
    def k(ids_hbm, tgt_hbm, ctx_hbm, out_t, out_c, idx_v, rows_v, sem_g):
        wid = lax.axis_index("s") * nc + lax.axis_index("c")
        base = wid * b_per_w
        pltpu.sync_copy(ids_hbm.at[pl.ds(base, b_per_w)],
                        idx_v.at[pl.ds(0, b_per_w)])
        out_slice = pl.ds(base, b_per_w)
        for tbl, out_hbm in ((tgt_hbm, out_t), (ctx_hbm, out_c)):

            def body(i, _):
                row = idx_v[pl.ds(i, LANES)][0]
                pltpu.async_copy(tbl.at[row], rows_v.at[i], sem_g)
                return 0

            lax.fori_loop(0, b_per_w, body, 0)
            pltpu.make_async_copy(
                out_hbm.at[out_slice], rows_v, sem_g).wait()
            pltpu.sync_copy(rows_v, out_hbm.at[out_slice])

    return k


@functools.lru_cache(maxsize=None)
def _make_tc(batch: int, num_nodes: int, d_emb: int):
    def body(ids_smem, tgt_ref, ctx_ref, out_t, out_c, sem):
        for tbl, out in ((tgt_ref, out_t), (ctx_ref, out_c)):

            def loop(i, _):
                row = ids_smem[i]
                pltpu.make_async_copy(tbl.at[row], out.at[i], sem).start()
                return 0

            lax.fori_loop(0, batch, loop, 0, unroll=8)
        pltpu.make_async_copy(
            tgt_ref.at[pl.ds(0, batch)], out_t, sem).wait()
        pltpu.make_async_copy(
            ctx_ref.at[pl.ds(0, batch)], out_c, sem).wait()

    return pl.pallas_call(
        body,
        grid_spec=pltpu.PrefetchScalarGridSpec(
            num_scalar_prefetch=1,
            grid=(1,),
            in_specs=[pl.BlockSpec(memory_space=pltpu.MemorySpace.HBM)] * 2,
            out_specs=[pl.BlockSpec(memory_space=pltpu.MemorySpace.HBM)] * 2,
            scratch_shapes=[pltpu.SemaphoreType.DMA],
        ),
        out_shape=(
            jax.ShapeDtypeStruct((batch, d_emb), jnp.float32),
            jax.ShapeDtypeStruct((batch, d_emb), jnp.float32),
        ),
    )


def kernel(ids, E_target, E_context):
    ids = ids.astype(jnp.int32)
    n, d = E_target.shape
    b = ids.shape[0]
    sc_k = _make_sc(b - TC_SHARE, n, d)
    tc_k = _make_tc(TC_SHARE, n, d)
    t_hi, c_hi = sc_k(ids[TC_SHARE:], E_target, E_context)
    t_lo, c_lo = tc_k(ids[:TC_SHARE], E_target, E_context)
    return (jnp.concatenate([t_lo, t_hi], axis=0),
            jnp.concatenate([c_lo, c_hi], axis=0))


# speedup vs baseline: 1.3223x; 1.0013x over previous
import functools

import jax
import jax.numpy as jnp
from jax import lax
from jax.experimental import pallas as pl
from jax.experimental.pallas import tpu as pltpu
from jax.experimental.pallas import tpu_sc as plsc

LANES = 16
TC_SHARE = 6144


@functools.lru_cache(maxsize=None)
def _make_sc(batch: int, num_nodes: int, d_emb: int):
    info = plsc.get_sparse_core_info()
    nc, ns = info.num_cores, info.num_subcores
    nw = nc * ns
    b_per_w = batch // nw
    mesh = plsc.VectorSubcoreMesh(core_axis_name="c", subcore_axis_name="s")

    @functools.partial(
        pl.kernel,
        mesh=mesh,
        out_type=(
            jax.ShapeDtypeStruct((batch, d_emb), jnp.float32),
            jax.ShapeDtypeStruct((batch, d_emb), jnp.float32),
        ),
        scratch_types=[
            pltpu.VMEM((b_per_w + LANES,), jnp.int32),
            pltpu.VMEM((b_per_w, d_emb), jnp.float32),
            pltpu.SemaphoreType.DMA,
        ],
        compiler_params=pltpu.CompilerParams(use_tc_tiling_on_sc=True,
                                             skip_device_barrier=True),
        cost_estimate=pl.CostEstimate(
            flops=0, transcendentals=0, bytes_accessed=1_000_000_000),
    )
    def k(ids_hbm, tgt_hbm, ctx_hbm, out_t, out_c, idx_v, rows_v, sem_g):
        wid = lax.axis_index("s") * nc + lax.axis_index("c")
        base = wid * b_per_w
        pltpu.sync_copy(ids_hbm.at[pl.ds(base, b_per_w)],
                        idx_v.at[pl.ds(0, b_per_w)])
        out_slice = pl.ds(base, b_per_w)
        for tbl, out_hbm in ((tgt_hbm, out_t), (ctx_hbm, out_c)):

            def body(i, _):
                row = idx_v[pl.ds(i, LANES)][0]
                pltpu.async_copy(tbl.at[row], rows_v.at[i], sem_g)
                return 0

            lax.fori_loop(0, b_per_w, body, 0)
            pltpu.make_async_copy(
                out_hbm.at[out_slice], rows_v, sem_g).wait()
            pltpu.sync_copy(rows_v, out_hbm.at[out_slice])

    return k


@functools.lru_cache(maxsize=None)
def _make_tc(batch: int, num_nodes: int, d_emb: int):
    def body(ids_smem, tgt_ref, ctx_ref, out_t, out_c, sem):
        for tbl, out in ((tgt_ref, out_t), (ctx_ref, out_c)):

            def loop(i, _):
                row = ids_smem[i]
                pltpu.make_async_copy(tbl.at[row], out.at[i], sem).start()
                return 0

            lax.fori_loop(0, batch, loop, 0, unroll=8)
        pltpu.make_async_copy(
            tgt_ref.at[pl.ds(0, batch)], out_t, sem).wait()
        pltpu.make_async_copy(
            ctx_ref.at[pl.ds(0, batch)], out_c, sem).wait()

    return pl.pallas_call(
        body,
        grid_spec=pltpu.PrefetchScalarGridSpec(
            num_scalar_prefetch=1,
            grid=(1,),
            in_specs=[pl.BlockSpec(memory_space=pltpu.MemorySpace.HBM)] * 2,
            out_specs=[pl.BlockSpec(memory_space=pltpu.MemorySpace.HBM)] * 2,
            scratch_shapes=[pltpu.SemaphoreType.DMA],
        ),
        out_shape=(
            jax.ShapeDtypeStruct((batch, d_emb), jnp.float32),
            jax.ShapeDtypeStruct((batch, d_emb), jnp.float32),
        ),
    )


def kernel(ids, E_target, E_context):
    ids = ids.astype(jnp.int32)
    n, d = E_target.shape
    b = ids.shape[0]
    sc_k = _make_sc(b - TC_SHARE, n, d)
    tc_k = _make_tc(TC_SHARE, n, d)
    t_hi, c_hi = sc_k(ids[TC_SHARE:], E_target, E_context)
    t_lo, c_lo = tc_k(ids[:TC_SHARE], E_target, E_context)
    return (jnp.concatenate([t_lo, t_hi], axis=0),
            jnp.concatenate([c_lo, c_hi], axis=0))
